# baseline (speedup 1.0000x reference)
import functools

import jax
import jax.numpy as jnp
from jax import lax
from jax.experimental import pallas as pl
from jax.experimental.pallas import tpu as pltpu
from jax.experimental.pallas import tpu_sc as plsc

D = 1024
TOTAL = 4 * 8192
N_SC = 24576
NW = 32
B_PER_W = N_SC // NW
CHUNK = 32
NBUF = 2
N_CHUNKS = B_PER_W // CHUNK
N_OUTER = N_CHUNKS // NBUF


def _make_gather():
    mesh = plsc.VectorSubcoreMesh(core_axis_name="c", subcore_axis_name="s")

    @functools.partial(
        pl.kernel,
        mesh=mesh,
        out_type=jax.ShapeDtypeStruct((N_SC, D), jnp.float32),
        scratch_types=[
            pltpu.VMEM((B_PER_W,), jnp.int32),
            pltpu.VMEM((NBUF, CHUNK, D), jnp.float32),
            pltpu.SemaphoreType.DMA((NBUF,)),
            pltpu.SemaphoreType.DMA((NBUF,)),
        ],
    )
    def gather_kernel(idx_hbm, table_hbm, out_hbm, idx_v, rows_v, gsem, osem):
        wid = lax.axis_index("s") * 2 + lax.axis_index("c")
        base = wid * B_PER_W
        pltpu.sync_copy(idx_hbm.at[pl.ds(base, B_PER_W)], idx_v)

        def gather_chunk(c, b):
            off = pl.multiple_of(c * CHUNK, CHUNK)
            return pltpu.make_async_copy(
                table_hbm.at[idx_v.at[pl.ds(off, CHUNK)]],
                rows_v.at[b],
                gsem.at[b],
            )

        def out_chunk(c, b):
            off = pl.multiple_of(c * CHUNK, CHUNK)
            return pltpu.make_async_copy(
                rows_v.at[b],
                out_hbm.at[pl.ds(base + off, CHUNK)],
                osem.at[b],
            )

        LOOK = NBUF // 2

        for b in range(LOOK):
            gather_chunk(b, b).start()

        def body(g, carry):
            for b in range(NBUF):
                c = g * NBUF + b
                bn = (b + LOOK) % NBUF
                if b + LOOK < NBUF:
                    @pl.when(g > 0)
                    def _():
                        out_chunk(c + LOOK - NBUF, bn).wait()

                    gather_chunk(c + LOOK, bn).start()
                else:
                    out_chunk(c + LOOK - NBUF, bn).wait()

                    @pl.when(c + LOOK < N_CHUNKS)
                    def _():
                        gather_chunk(c + LOOK, bn).start()

                gather_chunk(c, b).wait()
                out_chunk(c, b).start()

            return carry

        lax.fori_loop(0, N_OUTER, body, 0)

        for b in range(LOOK):
            c = N_CHUNKS - LOOK + b
            out_chunk(c, (NBUF - LOOK + b) % NBUF).wait()

    return gather_kernel


_gather = _make_gather()


@jax.jit
def kernel(position_ids, PosEnc):
    idx = position_ids.reshape(TOTAL).astype(jnp.int32)
    sc_out = _gather(idx[:N_SC], PosEnc)
    tc_out = jnp.take(PosEnc, idx[N_SC:], axis=0)
    out = jnp.concatenate([sc_out, tc_out], axis=0)
    return out.reshape(position_ids.shape + (D,))


# speedup vs baseline: 1.9933x; 1.9933x over previous
import functools

import jax
import jax.numpy as jnp
from jax import lax
from jax.experimental import pallas as pl
from jax.experimental.pallas import tpu as pltpu
from jax.experimental.pallas import tpu_sc as plsc

D = 1024
TOTAL = 4 * 8192
NW = 32
B_PER_W = TOTAL // NW
CHUNK = 16
NBUF = 4
LOOK = 1
N_CHUNKS = B_PER_W // CHUNK
N_OUTER = N_CHUNKS // NBUF
DRAIN = NBUF - LOOK


def _make_gather():
    mesh = plsc.VectorSubcoreMesh(core_axis_name="c", subcore_axis_name="s")

    @functools.partial(
        pl.kernel,
        mesh=mesh,
        out_type=jax.ShapeDtypeStruct((TOTAL, D), jnp.float32),
        scratch_types=[
            pltpu.VMEM((B_PER_W,), jnp.int32),
            pltpu.VMEM((NBUF, CHUNK, D), jnp.float32),
            pltpu.SemaphoreType.DMA((NBUF,)),
            pltpu.SemaphoreType.DMA((NBUF,)),
        ],
    )
    def gather_kernel(idx_hbm, table_hbm, out_hbm, idx_v, rows_v, gsem, osem):
        wid = lax.axis_index("s") * 2 + lax.axis_index("c")
        base = wid * B_PER_W
        pltpu.sync_copy(idx_hbm.at[pl.ds(base, B_PER_W)], idx_v)

        def gather_chunk(c, b):
            off = pl.multiple_of(c * CHUNK, CHUNK)
            return pltpu.make_async_copy(
                table_hbm.at[idx_v.at[pl.ds(off, CHUNK)]],
                rows_v.at[b],
                gsem.at[b],
            )

        def out_chunk(c, b):
            off = pl.multiple_of(c * CHUNK, CHUNK)
            return pltpu.make_async_copy(
                rows_v.at[b],
                out_hbm.at[pl.ds(base + off, CHUNK)],
                osem.at[b],
            )

        for b in range(LOOK):
            gather_chunk(b, b).start()

        def body(g, carry):
            for b in range(NBUF):
                c = g * NBUF + b
                bn = (b + LOOK) % NBUF
                if b + LOOK < NBUF:
                    @pl.when(g > 0)
                    def _():
                        out_chunk(c + LOOK - NBUF, bn).wait()

                    gather_chunk(c + LOOK, bn).start()
                else:
                    out_chunk(c + LOOK - NBUF, bn).wait()

                    @pl.when(c + LOOK < N_CHUNKS)
                    def _():
                        gather_chunk(c + LOOK, bn).start()

                gather_chunk(c, b).wait()
                out_chunk(c, b).start()

            return carry

        lax.fori_loop(0, N_OUTER, body, 0)

        for k in range(DRAIN):
            c = N_CHUNKS - DRAIN + k
            out_chunk(c, c % NBUF).wait()

    return gather_kernel


_gather = _make_gather()


@jax.jit
def kernel(position_ids, PosEnc):
    idx = position_ids.reshape(TOTAL).astype(jnp.int32)
    out = _gather(idx, PosEnc)
    return out.reshape(position_ids.shape + (D,))


# speedup vs baseline: 2.0271x; 1.0170x over previous
import functools

import jax
import jax.numpy as jnp
from jax import lax
from jax.experimental import pallas as pl
from jax.experimental.pallas import tpu as pltpu
from jax.experimental.pallas import tpu_sc as plsc

D = 1024
TOTAL = 4 * 8192
NW = 32
NS = 16
B_PER_W = TOTAL // NW
CHUNK = 16
NBUF = 4
LOOK = 2
N_CHUNKS = B_PER_W // CHUNK
N_OUTER = N_CHUNKS // NBUF


def _make_gather():
    mesh = plsc.VectorSubcoreMesh(core_axis_name="c", subcore_axis_name="s")

    @functools.partial(
        pl.kernel,
        mesh=mesh,
        out_type=jax.ShapeDtypeStruct((TOTAL, D), jnp.float32),
        scratch_types=[
            pltpu.VMEM((B_PER_W,), jnp.int32),
            pltpu.VMEM((NBUF, CHUNK, D), jnp.float32),
            pltpu.VMEM_SHARED((NS, 2, CHUNK, D), jnp.float32),
            pltpu.SemaphoreType.DMA((NBUF,)),
            pltpu.SemaphoreType.DMA((NBUF,)),
            pltpu.SemaphoreType.DMA((2,)),
        ],
    )
    def gather_kernel(idx_hbm, table_hbm, out_hbm,
                      idx_v, rows_v, spm, gsem, osem, osem2):
        sid = lax.axis_index("s")
        wid = sid * 2 + lax.axis_index("c")
        base = wid * B_PER_W
        pltpu.sync_copy(idx_hbm.at[pl.ds(base, B_PER_W)], idx_v)

        def gather_chunk(c, b):
            off = pl.multiple_of(c * CHUNK, CHUNK)
            return pltpu.make_async_copy(
                table_hbm.at[idx_v.at[pl.ds(off, CHUNK)]],
                rows_v.at[b],
                gsem.at[b],
            )

        def out_direct(c, b):
            off = pl.multiple_of(c * CHUNK, CHUNK)
            return pltpu.make_async_copy(
                rows_v.at[b],
                out_hbm.at[pl.ds(base + off, CHUNK)],
                osem.at[b],
            )

        def out_spmem(c, j):
            off = pl.multiple_of(c * CHUNK, CHUNK)
            return pltpu.make_async_copy(
                spm.at[sid, j],
                out_hbm.at[pl.ds(base + off, CHUNK)],
                osem2.at[j],
            )

        for b in range(LOOK):
            gather_chunk(b, b).start()

        def body(g, carry):
            for b in range(NBUF):
                c = g * NBUF + b
                bn = (b + LOOK) % NBUF
                if bn % 2 == 0:
                    if b + LOOK < NBUF:
                        @pl.when(g > 0)
                        def _():
                            out_direct(c + LOOK - NBUF, bn).wait()
                    else:
                        out_direct(c + LOOK - NBUF, bn).wait()

                if b + LOOK < NBUF:
                    gather_chunk(c + LOOK, bn).start()
                else:
                    @pl.when(c + LOOK < N_CHUNKS)
                    def _():
                        gather_chunk(c + LOOK, bn).start()

                gather_chunk(c, b).wait()
                if b % 2 == 0:
                    out_direct(c, b).start()
                else:
                    j = (b - 1) // 2
                    @pl.when(g > 0)
                    def _():
                        out_spmem(c - NBUF, j).wait()

                    pltpu.sync_copy(rows_v.at[b], spm.at[sid, j])
                    out_spmem(c, j).start()

            return carry

        lax.fori_loop(0, N_OUTER, body, 0)

        out_direct(N_CHUNKS - 2, (N_CHUNKS - 2) % NBUF).wait()
        out_spmem(N_CHUNKS - 3, 0).wait()
        out_spmem(N_CHUNKS - 1, 1).wait()

    return gather_kernel


_gather = _make_gather()


@jax.jit
def kernel(position_ids, PosEnc):
    idx = position_ids.reshape(TOTAL).astype(jnp.int32)
    out = _gather(idx, PosEnc)
    return out.reshape(position_ids.shape + (D,))


# speedup vs baseline: 2.0342x; 1.0035x over previous
import functools

import jax
import jax.numpy as jnp
from jax import lax
from jax.experimental import pallas as pl
from jax.experimental.pallas import tpu as pltpu
from jax.experimental.pallas import tpu_sc as plsc

D = 1024
TOTAL = 4 * 8192
NW = 32
NS = 16
B_PER_W = TOTAL // NW
CHUNK = 16
NBUF = 4
LOOK = 2
N_CHUNKS = B_PER_W // CHUNK
N_OUTER = N_CHUNKS // NBUF


def _make_gather():
    mesh = plsc.VectorSubcoreMesh(core_axis_name="c", subcore_axis_name="s")

    @functools.partial(
        pl.kernel,
        mesh=mesh,
        out_type=jax.ShapeDtypeStruct((TOTAL, D), jnp.float32),
        scratch_types=[
            pltpu.VMEM((B_PER_W,), jnp.int32),
            pltpu.VMEM((NBUF, CHUNK, D), jnp.float32),
            pltpu.VMEM_SHARED((NS, 3, CHUNK, D), jnp.float32),
            pltpu.SemaphoreType.DMA((NBUF,)),
            pltpu.SemaphoreType.DMA,
            pltpu.SemaphoreType.DMA((3,)),
            pltpu.SemaphoreType.DMA((3,)),
        ],
    )
    def gather_kernel(idx_hbm, table_hbm, out_hbm,
                      idx_v, rows_v, spm, gsem, osem, xsem, dsem):
        sid = lax.axis_index("s")
        wid = sid * 2 + lax.axis_index("c")
        base = wid * B_PER_W
        pltpu.sync_copy(idx_hbm.at[pl.ds(base, B_PER_W)], idx_v)

        def gather_chunk(c, b):
            off = pl.multiple_of(c * CHUNK, CHUNK)
            return pltpu.make_async_copy(
                table_hbm.at[idx_v.at[pl.ds(off, CHUNK)]],
                rows_v.at[b],
                gsem.at[b],
            )

        def out_direct(c):
            off = pl.multiple_of(c * CHUNK, CHUNK)
            return pltpu.make_async_copy(
                rows_v.at[0],
                out_hbm.at[pl.ds(base + off, CHUNK)],
                osem,
            )

        def crossbar(b, j):
            return pltpu.make_async_copy(
                rows_v.at[b], spm.at[sid, j], xsem.at[j]
            )

        def drain(c, j):
            off = pl.multiple_of(c * CHUNK, CHUNK)
            return pltpu.make_async_copy(
                spm.at[sid, j],
                out_hbm.at[pl.ds(base + off, CHUNK)],
                dsem.at[j],
            )

        for b in range(LOOK):
            gather_chunk(b, b).start()

        def body(g, carry):
            for b in range(NBUF):
                c = g * NBUF + b

                pb = (b - 2) % NBUF
                if pb != 0:
                    pj = pb - 1
                    if b < 2:
                        @pl.when(g > 0)
                        def _():
                            crossbar(pb, pj).wait()
                            drain(c - 2, pj).start()
                    else:
                        crossbar(pb, pj).wait()
                        drain(c - 2, pj).start()

                bn = (b + LOOK) % NBUF
                if bn == 0:
                    out_direct(c + LOOK - NBUF).wait()
                if b + LOOK < NBUF:
                    gather_chunk(c + LOOK, bn).start()
                else:
                    @pl.when(c + LOOK < N_CHUNKS)
                    def _():
                        gather_chunk(c + LOOK, bn).start()

                gather_chunk(c, b).wait()
                if b == 0:
                    out_direct(c).start()
                else:
                    j = b - 1

                    @pl.when(g > 0)
                    def _():
                        drain(c - NBUF, j).wait()

                    crossbar(b, j).start()

            return carry

        lax.fori_loop(0, N_OUTER, body, 0)

        last = N_CHUNKS - 1
        crossbar(2, 1).wait()
        drain(last - 1, 1).start()
        crossbar(3, 2).wait()
        drain(last, 2).start()
        drain(last - 2, 0).wait()
        drain(last - 1, 1).wait()
        drain(last, 2).wait()

    return gather_kernel


_gather = _make_gather()


@jax.jit
def kernel(position_ids, PosEnc):
    idx = position_ids.reshape(TOTAL).astype(jnp.int32)
    out = _gather(idx, PosEnc)
    return out.reshape(position_ids.shape + (D,))
